# baseline (speedup 1.0000x reference)
import functools

import jax
import jax.numpy as jnp
from jax import lax
from jax.experimental import pallas as pl
from jax.experimental.pallas import tpu as pltpu
from jax.experimental.pallas import tpu_sc as plsc

_EPS = 1e-09


def _mlp_body(x_ref, w1_ref, b1_ref, w2_ref, b2_ref, w3_ref, b3_ref, out_ref):
    xb = x_ref[...]
    xh = xb.astype(jnp.bfloat16)
    xl = (xb - xh.astype(jnp.float32)).astype(jnp.bfloat16)
    w1h = w1_ref[...].astype(jnp.bfloat16)
    h = (jnp.dot(xh, w1h, preferred_element_type=jnp.float32)
         + jnp.dot(xl, w1h, preferred_element_type=jnp.float32)) + b1_ref[...]
    h = h * (1.0 / (1.0 + jnp.exp(-h)))
    hh = h.astype(jnp.bfloat16)
    hl = (h - hh.astype(jnp.float32)).astype(jnp.bfloat16)
    w2h = w2_ref[...].astype(jnp.bfloat16)
    h = (jnp.dot(hh, w2h, preferred_element_type=jnp.float32)
         + jnp.dot(hl, w2h, preferred_element_type=jnp.float32)) + b2_ref[...]
    h = h * (1.0 / (1.0 + jnp.exp(-h)))
    s = jnp.sum(h * w3_ref[...], axis=1) + b3_ref[0]
    out_ref[...] = jnp.exp(s)[None, None, :]


def _run_mlp(x, W1, b1, W2, b2, W3, b3, block_m):
    n, d = x.shape
    h = W1.shape[1]
    grid = (n // block_m,)
    return pl.pallas_call(
        _mlp_body,
        grid=grid,
        in_specs=[
            pl.BlockSpec((block_m, d), lambda i: (i, 0)),
            pl.BlockSpec((d, h), lambda i: (0, 0)),
            pl.BlockSpec((1, h), lambda i: (0, 0)),
            pl.BlockSpec((h, h), lambda i: (0, 0)),
            pl.BlockSpec((1, h), lambda i: (0, 0)),
            pl.BlockSpec((1, h), lambda i: (0, 0)),
            pl.BlockSpec(memory_space=pltpu.SMEM),
        ],
        out_specs=pl.BlockSpec((1, 1, block_m), lambda i: (i, 0, 0)),
        out_shape=jax.ShapeDtypeStruct((n // block_m, 1, block_m), jnp.float32),
    )(x, W1, b1.reshape(1, h), W2, b2.reshape(1, h), W3.reshape(1, h), b3)


def _sc_segment_body(es_hbm, y_hbm, g_hbm, a_hbm, q_hbm,
                     es_v, y_v, et_v, g_v, dens_loc, dent_loc,
                     a_v, q_v, zero_v, dens_sh, dent_sh):
    cid = lax.axis_index("c")
    sid = lax.axis_index("s")
    row0 = sid * 32

    pltpu.sync_copy(es_hbm.at[pl.ds(row0, 32)], es_v)
    pltpu.sync_copy(y_hbm.at[pl.ds(row0, 32)], y_v)
    pltpu.sync_copy(g_hbm.at[pl.ds(row0, 32)], g_v)

    def _expy_row(r, _):
        for j in range(8):
            et_v[r, pl.ds(16 * j, 16)] = jnp.exp(y_v[r, pl.ds(16 * j, 16)])
        return 0
    lax.fori_loop(0, 32, _expy_row, 0)

    @pl.when(sid == 0)
    def _zero():
        def _z(i, _):
            zero_v[pl.ds(16 * i, 16)] = jnp.zeros((16,), jnp.float32)
            return 0
        lax.fori_loop(0, 128, _z, 0)
        pltpu.sync_copy(zero_v, dens_sh)
        pltpu.sync_copy(zero_v, dent_sh)

    plsc.subcore_barrier()

    def _scat(j, _):
        pltpu.sync_copy(es_v.at[j], dens_sh.at[g_v.at[j]], add=True)
        pltpu.sync_copy(et_v.at[j], dent_sh.at[g_v.at[j]], add=True)
        return 0
    lax.fori_loop(0, 32, _scat, 0)

    plsc.subcore_barrier()

    pltpu.sync_copy(dens_sh, dens_loc)
    pltpu.sync_copy(dent_sh, dent_loc)
    lr0 = cid * 16

    def _gath(rr, _):
        lr = lr0 + rr
        for j in range(8):
            c = pl.ds(16 * j, 16)
            gv = g_v[lr, c]
            dsv = plsc.load_gather(dens_loc, [gv])
            dtv = plsc.load_gather(dent_loc, [gv])
            a_v[rr, c] = es_v[lr, c] / (dsv + _EPS) + _EPS
            q_v[rr, c] = et_v[lr, c] / (dtv + _EPS)
        return 0
    lax.fori_loop(0, 16, _gath, 0)

    orow = row0 + lr0
    pltpu.sync_copy(a_v, a_hbm.at[pl.ds(orow, 16)])
    pltpu.sync_copy(q_v, q_hbm.at[pl.ds(orow, 16)])


def _run_sc_segment(exp_s2, y2, g2, num_groups):
    rows = exp_s2.shape[0]
    mesh = plsc.VectorSubcoreMesh(core_axis_name="c", subcore_axis_name="s")
    f32 = jnp.float32
    kern = pl.kernel(
        _sc_segment_body,
        compiler_params=pltpu.CompilerParams(needs_layout_passes=False),
        out_type=(
            jax.ShapeDtypeStruct((rows, 128), f32),
            jax.ShapeDtypeStruct((rows, 128), f32),
        ),
        mesh=mesh,
        scratch_types=[
            pltpu.VMEM((32, 128), f32),
            pltpu.VMEM((32, 128), f32),
            pltpu.VMEM((32, 128), f32),
            pltpu.VMEM((32, 128), jnp.int32),
            pltpu.VMEM((num_groups,), f32),
            pltpu.VMEM((num_groups,), f32),
            pltpu.VMEM((16, 128), f32),
            pltpu.VMEM((16, 128), f32),
            pltpu.VMEM((num_groups,), f32),
            pltpu.VMEM_SHARED((num_groups,), f32),
            pltpu.VMEM_SHARED((num_groups,), f32),
        ],
    )
    return kern(exp_s2, y2, g2)


def _fin_body(a_ref, q_ref, gc_ref, gp_ref, out_ref):
    tot = jnp.sum(-q_ref[...] * jnp.log(a_ref[...]))
    nb = jnp.sum((gc_ref[...] != gp_ref[...]).astype(jnp.float32))
    out_ref[0, 0] = tot / jnp.maximum(nb, 1.0)


def _run_finalize(a2, q2, gc2, gp2):
    return pl.pallas_call(
        _fin_body,
        in_specs=[pl.BlockSpec(a2.shape, lambda: (0, 0))] * 4,
        out_specs=pl.BlockSpec(memory_space=pltpu.SMEM),
        out_shape=jax.ShapeDtypeStruct((1, 1), jnp.float32),
    )(a2, q2, gc2, gp2)


def kernel(x, y, g, W1, b1, W2, b2, W3, b3):
    n = x.shape[0]
    num_groups = 2048
    rows = n // 128

    exp_s = _run_mlp(x, W1, b1, W2, b2, W3, b3, block_m=2048)
    exp_s2 = exp_s.reshape(rows, 128)
    y2 = y.reshape(rows, 128)
    g2 = g.reshape(rows, 128)

    a2, q2 = _run_sc_segment(exp_s2, y2, g2, num_groups)

    g_prev = jnp.concatenate([g[:1] - 1, g[:-1]])
    loss = _run_finalize(a2, q2, g2, g_prev.reshape(rows, 128))
    return loss[0, 0]


# speedup vs baseline: 1.2767x; 1.2767x over previous
import functools

import jax
import jax.numpy as jnp
from jax import lax
from jax.experimental import pallas as pl
from jax.experimental.pallas import tpu as pltpu
from jax.experimental.pallas import tpu_sc as plsc

_EPS = 1e-09


def _mlp_body(x_ref, w1_ref, b1_ref, w2_ref, b2_ref, w3_ref, b3_ref, out_ref):
    xh = x_ref[...].astype(jnp.bfloat16)
    w1h = w1_ref[...].astype(jnp.bfloat16)
    h = jnp.dot(xh, w1h, preferred_element_type=jnp.float32) + b1_ref[...]
    h = h * (1.0 / (1.0 + jnp.exp(-h)))
    hh = h.astype(jnp.bfloat16)
    w2h = w2_ref[...].astype(jnp.bfloat16)
    h = jnp.dot(hh, w2h, preferred_element_type=jnp.float32) + b2_ref[...]
    h = h * (1.0 / (1.0 + jnp.exp(-h)))
    s = jnp.sum(h * w3_ref[...], axis=1) + b3_ref[0]
    out_ref[...] = jnp.exp(s)[None, None, :]


def _run_mlp(x, W1, b1, W2, b2, W3, b3, block_m):
    n, d = x.shape
    h = W1.shape[1]
    grid = (n // block_m,)
    return pl.pallas_call(
        _mlp_body,
        grid=grid,
        in_specs=[
            pl.BlockSpec((block_m, d), lambda i: (i, 0)),
            pl.BlockSpec((d, h), lambda i: (0, 0)),
            pl.BlockSpec((1, h), lambda i: (0, 0)),
            pl.BlockSpec((h, h), lambda i: (0, 0)),
            pl.BlockSpec((1, h), lambda i: (0, 0)),
            pl.BlockSpec((1, h), lambda i: (0, 0)),
            pl.BlockSpec(memory_space=pltpu.SMEM),
        ],
        out_specs=pl.BlockSpec((1, 1, block_m), lambda i: (i, 0, 0)),
        out_shape=jax.ShapeDtypeStruct((n // block_m, 1, block_m), jnp.float32),
    )(x, W1, b1.reshape(1, h), W2, b2.reshape(1, h), W3.reshape(1, h), b3)


def _sc_segment_body(es_hbm, y_hbm, g_hbm, a_hbm, q_hbm,
                     es_v, y_v, et_v, g_v, dens_loc, dent_loc,
                     a_v, q_v, zero_v, dens_sh, dent_sh):
    cid = lax.axis_index("c")
    sid = lax.axis_index("s")
    row0 = sid * 32

    pltpu.sync_copy(es_hbm.at[pl.ds(row0, 32)], es_v)
    pltpu.sync_copy(y_hbm.at[pl.ds(row0, 32)], y_v)
    pltpu.sync_copy(g_hbm.at[pl.ds(row0, 32)], g_v)

    def _expy_row(r, _):
        for j in range(8):
            et_v[r, pl.ds(16 * j, 16)] = jnp.exp(y_v[r, pl.ds(16 * j, 16)])
        return 0
    lax.fori_loop(0, 32, _expy_row, 0)

    @pl.when(sid == 0)
    def _zero():
        def _z(i, _):
            zero_v[pl.ds(16 * i, 16)] = jnp.zeros((16,), jnp.float32)
            return 0
        lax.fori_loop(0, 128, _z, 0)
        pltpu.sync_copy(zero_v, dens_sh)
        pltpu.sync_copy(zero_v, dent_sh)

    plsc.subcore_barrier()

    def _scat(j, _):
        pltpu.sync_copy(es_v.at[j], dens_sh.at[g_v.at[j]], add=True)
        pltpu.sync_copy(et_v.at[j], dent_sh.at[g_v.at[j]], add=True)
        return 0
    lax.fori_loop(0, 32, _scat, 0)

    plsc.subcore_barrier()

    pltpu.sync_copy(dens_sh, dens_loc)
    pltpu.sync_copy(dent_sh, dent_loc)
    lr0 = cid * 16

    def _gath(rr, _):
        lr = lr0 + rr
        for j in range(8):
            c = pl.ds(16 * j, 16)
            gv = g_v[lr, c]
            dsv = plsc.load_gather(dens_loc, [gv])
            dtv = plsc.load_gather(dent_loc, [gv])
            a_v[rr, c] = es_v[lr, c] / (dsv + _EPS) + _EPS
            q_v[rr, c] = et_v[lr, c] / (dtv + _EPS)
        return 0
    lax.fori_loop(0, 16, _gath, 0)

    orow = row0 + lr0
    pltpu.sync_copy(a_v, a_hbm.at[pl.ds(orow, 16)])
    pltpu.sync_copy(q_v, q_hbm.at[pl.ds(orow, 16)])


def _run_sc_segment(exp_s2, y2, g2, num_groups):
    rows = exp_s2.shape[0]
    mesh = plsc.VectorSubcoreMesh(core_axis_name="c", subcore_axis_name="s")
    f32 = jnp.float32
    kern = pl.kernel(
        _sc_segment_body,
        compiler_params=pltpu.CompilerParams(needs_layout_passes=False),
        out_type=(
            jax.ShapeDtypeStruct((rows, 128), f32),
            jax.ShapeDtypeStruct((rows, 128), f32),
        ),
        mesh=mesh,
        scratch_types=[
            pltpu.VMEM((32, 128), f32),
            pltpu.VMEM((32, 128), f32),
            pltpu.VMEM((32, 128), f32),
            pltpu.VMEM((32, 128), jnp.int32),
            pltpu.VMEM((num_groups,), f32),
            pltpu.VMEM((num_groups,), f32),
            pltpu.VMEM((16, 128), f32),
            pltpu.VMEM((16, 128), f32),
            pltpu.VMEM((num_groups,), f32),
            pltpu.VMEM_SHARED((num_groups,), f32),
            pltpu.VMEM_SHARED((num_groups,), f32),
        ],
    )
    return kern(exp_s2, y2, g2)


def _fin_body(a_ref, q_ref, gc_ref, gp_ref, out_ref):
    tot = jnp.sum(-q_ref[...] * jnp.log(a_ref[...]))
    nb = jnp.sum((gc_ref[...] != gp_ref[...]).astype(jnp.float32))
    out_ref[0, 0] = tot / jnp.maximum(nb, 1.0)


def _run_finalize(a2, q2, gc2, gp2):
    return pl.pallas_call(
        _fin_body,
        in_specs=[pl.BlockSpec(a2.shape, lambda: (0, 0))] * 4,
        out_specs=pl.BlockSpec(memory_space=pltpu.SMEM),
        out_shape=jax.ShapeDtypeStruct((1, 1), jnp.float32),
    )(a2, q2, gc2, gp2)


def kernel(x, y, g, W1, b1, W2, b2, W3, b3):
    n = x.shape[0]
    num_groups = 2048
    rows = n // 128

    exp_s = _run_mlp(x, W1, b1, W2, b2, W3, b3, block_m=2048)
    exp_s2 = exp_s.reshape(rows, 128)
    y2 = y.reshape(rows, 128)
    g2 = g.reshape(rows, 128)

    a2, q2 = _run_sc_segment(exp_s2, y2, g2, num_groups)

    g_prev = jnp.concatenate([g[:1] - 1, g[:-1]])
    loss = _run_finalize(a2, q2, g2, g_prev.reshape(rows, 128))
    return loss[0, 0]


# speedup vs baseline: 2.2068x; 1.7285x over previous
import functools

import jax
import jax.numpy as jnp
from jax import lax
from jax.experimental import pallas as pl
from jax.experimental.pallas import tpu as pltpu
from jax.experimental.pallas import tpu_sc as plsc

_EPS = 1e-09


def _mlp_body(x_ref, w1_ref, b1_ref, w2_ref, b2_ref, w3_ref, b3_ref, out_ref):
    xh = x_ref[...].astype(jnp.bfloat16)
    w1h = w1_ref[...].astype(jnp.bfloat16)
    h = jnp.dot(xh, w1h, preferred_element_type=jnp.float32) + b1_ref[...]
    h = h * (1.0 / (1.0 + jnp.exp(-h)))
    hh = h.astype(jnp.bfloat16)
    w2h = w2_ref[...].astype(jnp.bfloat16)
    h = jnp.dot(hh, w2h, preferred_element_type=jnp.float32) + b2_ref[...]
    h = h * (1.0 / (1.0 + jnp.exp(-h)))
    s = lax.dot_general(
        w3_ref[...].astype(jnp.bfloat16), h.astype(jnp.bfloat16),
        dimension_numbers=(((1,), (1,)), ((), ())),
        preferred_element_type=jnp.float32,
    ) + b3_ref[0]
    out_ref[...] = jnp.exp(s)[None]


def _run_mlp(x, W1, b1, W2, b2, W3, b3, block_m):
    n, d = x.shape
    h = W1.shape[1]
    grid = (n // block_m,)
    return pl.pallas_call(
        _mlp_body,
        grid=grid,
        in_specs=[
            pl.BlockSpec((block_m, d), lambda i: (i, 0)),
            pl.BlockSpec((d, h), lambda i: (0, 0)),
            pl.BlockSpec((1, h), lambda i: (0, 0)),
            pl.BlockSpec((h, h), lambda i: (0, 0)),
            pl.BlockSpec((1, h), lambda i: (0, 0)),
            pl.BlockSpec((1, h), lambda i: (0, 0)),
            pl.BlockSpec(memory_space=pltpu.SMEM),
        ],
        out_specs=pl.BlockSpec((1, 1, block_m), lambda i: (i, 0, 0)),
        out_shape=jax.ShapeDtypeStruct((n // block_m, 1, block_m), jnp.float32),
    )(x, W1, b1.reshape(1, h), W2, b2.reshape(1, h), W3.reshape(1, h), b3)


def _sc_segment_body(es_hbm, y_hbm, g_hbm, a_hbm, q_hbm,
                     es_v, y_v, et_v, g_v, dens_loc, dent_loc,
                     a_v, q_v, zero_v, dens_sh, dent_sh):
    cid = lax.axis_index("c")
    sid = lax.axis_index("s")
    row0 = sid * 32

    pltpu.sync_copy(es_hbm.at[pl.ds(row0, 32)], es_v)
    pltpu.sync_copy(y_hbm.at[pl.ds(row0, 32)], y_v)
    pltpu.sync_copy(g_hbm.at[pl.ds(row0, 32)], g_v)

    def _expy_row(r, _):
        for j in range(8):
            et_v[r, pl.ds(16 * j, 16)] = jnp.exp(y_v[r, pl.ds(16 * j, 16)])
        return 0
    lax.fori_loop(0, 32, _expy_row, 0)

    @pl.when(sid == 0)
    def _zero():
        def _z(i, _):
            zero_v[pl.ds(16 * i, 16)] = jnp.zeros((16,), jnp.float32)
            return 0
        lax.fori_loop(0, 128, _z, 0)
        pltpu.sync_copy(zero_v, dens_sh)
        pltpu.sync_copy(zero_v, dent_sh)

    plsc.subcore_barrier()

    def _scat(j, _):
        pltpu.sync_copy(es_v.at[j], dens_sh.at[g_v.at[j]], add=True)
        pltpu.sync_copy(et_v.at[j], dent_sh.at[g_v.at[j]], add=True)
        return 0
    lax.fori_loop(0, 32, _scat, 0)

    plsc.subcore_barrier()

    pltpu.sync_copy(dens_sh, dens_loc)
    pltpu.sync_copy(dent_sh, dent_loc)
    lr0 = cid * 16

    def _gath(rr, _):
        lr = lr0 + rr
        for j in range(8):
            c = pl.ds(16 * j, 16)
            gv = g_v[lr, c]
            dsv = plsc.load_gather(dens_loc, [gv])
            dtv = plsc.load_gather(dent_loc, [gv])
            a_v[rr, c] = es_v[lr, c] / (dsv + _EPS) + _EPS
            q_v[rr, c] = et_v[lr, c] / (dtv + _EPS)
        return 0
    lax.fori_loop(0, 16, _gath, 0)

    orow = row0 + lr0
    pltpu.sync_copy(a_v, a_hbm.at[pl.ds(orow, 16)])
    pltpu.sync_copy(q_v, q_hbm.at[pl.ds(orow, 16)])


def _run_sc_segment(exp_s2, y2, g2, num_groups):
    rows = exp_s2.shape[0]
    mesh = plsc.VectorSubcoreMesh(core_axis_name="c", subcore_axis_name="s")
    f32 = jnp.float32
    kern = pl.kernel(
        _sc_segment_body,
        compiler_params=pltpu.CompilerParams(needs_layout_passes=False),
        out_type=(
            jax.ShapeDtypeStruct((rows, 128), f32),
            jax.ShapeDtypeStruct((rows, 128), f32),
        ),
        mesh=mesh,
        scratch_types=[
            pltpu.VMEM((32, 128), f32),
            pltpu.VMEM((32, 128), f32),
            pltpu.VMEM((32, 128), f32),
            pltpu.VMEM((32, 128), jnp.int32),
            pltpu.VMEM((num_groups,), f32),
            pltpu.VMEM((num_groups,), f32),
            pltpu.VMEM((16, 128), f32),
            pltpu.VMEM((16, 128), f32),
            pltpu.VMEM((num_groups,), f32),
            pltpu.VMEM_SHARED((num_groups,), f32),
            pltpu.VMEM_SHARED((num_groups,), f32),
        ],
    )
    return kern(exp_s2, y2, g2)


def _fin_body(a_ref, q_ref, gc_ref, gp_ref, out_ref):
    tot = jnp.sum(-q_ref[...] * jnp.log(a_ref[...]))
    nb = jnp.sum((gc_ref[...] != gp_ref[...]).astype(jnp.float32))
    out_ref[0, 0] = tot / jnp.maximum(nb, 1.0)


def _run_finalize(a2, q2, gc2, gp2):
    return pl.pallas_call(
        _fin_body,
        in_specs=[pl.BlockSpec(a2.shape, lambda: (0, 0))] * 4,
        out_specs=pl.BlockSpec(memory_space=pltpu.SMEM),
        out_shape=jax.ShapeDtypeStruct((1, 1), jnp.float32),
    )(a2, q2, gc2, gp2)


def kernel(x, y, g, W1, b1, W2, b2, W3, b3):
    n = x.shape[0]
    num_groups = 2048
    rows = n // 128

    exp_s = _run_mlp(x, W1, b1, W2, b2, W3, b3, block_m=2048)
    exp_s2 = exp_s.reshape(rows, 128)
    y2 = y.reshape(rows, 128)
    g2 = g.reshape(rows, 128)

    a2, q2 = _run_sc_segment(exp_s2, y2, g2, num_groups)

    g_prev = jnp.concatenate([g[:1] - 1, g[:-1]])
    loss = _run_finalize(a2, q2, g2, g_prev.reshape(rows, 128))
    return loss[0, 0]
